# baseline (speedup 1.0000x reference)
import functools

import jax
import jax.numpy as jnp
from jax import lax
from jax.experimental import pallas as pl
from jax.experimental.pallas import tpu as pltpu
from jax.experimental.pallas import tpu_sc as plsc

_EMBED = 16
_BATCH = 16384
_REGS = 0.0001
_NC, _NS, _L = 2, 16, 16
_NW = _NC * _NS
_BPW = _BATCH // _NW
_CH = 128
_NCH = _BPW // _CH

_mesh = plsc.VectorSubcoreMesh(core_axis_name="c", subcore_axis_name="s")


@functools.partial(
    pl.kernel,
    out_type=(
        jax.ShapeDtypeStruct((_BATCH,), jnp.float32),
        jax.ShapeDtypeStruct((_NW, _L), jnp.float32),
    ),
    mesh=_mesh,
    compiler_params=pltpu.CompilerParams(
        needs_layout_passes=False, use_tc_tiling_on_sc=False),
    scratch_types=(
        pltpu.VMEM((_NCH, _CH), jnp.int32),
        pltpu.VMEM((_NCH, _CH), jnp.int32),
        pltpu.VMEM((_NCH, _CH), jnp.int32),
        pltpu.VMEM((_BPW, _EMBED), jnp.float32),
        pltpu.VMEM((_BPW, _EMBED), jnp.float32),
        pltpu.VMEM((_BPW, _EMBED), jnp.float32),
        pltpu.VMEM((_BPW,), jnp.float32),
        pltpu.VMEM((_L,), jnp.float32),
        pltpu.SemaphoreType.DMA,
    ),
)
def _sc_gather_score(user, pos, neg, uemb, iemb, diff_out, acc_out,
                     uidx, pidx, nidx, urows, prows, nrows, diffv, accv, sem):
    wid = lax.axis_index("s") * _NC + lax.axis_index("c")
    base = wid * _BPW
    for j in range(_NCH):
        off = base + j * _CH
        pltpu.sync_copy(user.at[pl.ds(off, _CH)], uidx.at[j])
        pltpu.sync_copy(pos.at[pl.ds(off, _CH)], pidx.at[j])
        pltpu.sync_copy(neg.at[pl.ds(off, _CH)], nidx.at[j])
    copies = []
    for j in range(_NCH):
        dst = pl.ds(j * _CH, _CH)
        copies.append(pltpu.async_copy(uemb.at[uidx.at[j]], urows.at[dst], sem))
        copies.append(pltpu.async_copy(iemb.at[pidx.at[j]], prows.at[dst], sem))
        copies.append(pltpu.async_copy(iemb.at[nidx.at[j]], nrows.at[dst], sem))
    for c in copies:
        c.wait()

    def body(g, acc):
        rows = g * _L + jnp.arange(_L, dtype=jnp.int32)
        score = jnp.zeros((_L,), jnp.float32)
        for l in range(_EMBED):
            lv = jnp.full((_L,), l, jnp.int32)
            uc = plsc.load_gather(urows, [rows, lv])
            pc = plsc.load_gather(prows, [rows, lv])
            nc = plsc.load_gather(nrows, [rows, lv])
            score = score + uc * (pc - nc)
            acc = acc + uc * uc + pc * pc + nc * nc
        diffv[pl.ds(g * _L, _L)] = score
        return acc

    acc = lax.fori_loop(0, _BPW // _L, body, jnp.zeros((_L,), jnp.float32))
    accv[...] = acc
    pltpu.sync_copy(diffv, diff_out.at[pl.ds(base, _BPW)])
    pltpu.sync_copy(accv, acc_out.at[wid])


def _tc_finish_body(diff_ref, acc_ref, loss_ref, reg_ref):
    d = diff_ref[...]
    ls = jnp.minimum(d, 0.0) - jnp.log1p(jnp.exp(-jnp.abs(d)))
    loss_ref[0, 0] = -jnp.sum(ls) * (1.0 / _BATCH)
    reg_ref[0, 0] = (_REGS * 0.5 / _BATCH) * jnp.sum(acc_ref[...])


def _tc_finish(diff, acc):
    loss, reg = pl.pallas_call(
        _tc_finish_body,
        out_shape=(
            jax.ShapeDtypeStruct((1, 1), jnp.float32),
            jax.ShapeDtypeStruct((1, 1), jnp.float32),
        ),
        out_specs=(
            pl.BlockSpec(memory_space=pltpu.SMEM),
            pl.BlockSpec(memory_space=pltpu.SMEM),
        ),
    )(diff.reshape(_BATCH // 128, 128), acc)
    return loss[0, 0], reg[0, 0]


def kernel(user, pos, neg, user_embedding, item_embedding):
    diff, acc = _sc_gather_score(user, pos, neg, user_embedding, item_embedding)
    loss, reg_loss = _tc_finish(diff, acc)
    return (loss, reg_loss)
